# baseline (speedup 1.0000x reference)
import functools

import jax
import jax.numpy as jnp
from jax import lax
from jax.experimental import pallas as pl
from jax.experimental.pallas import tpu as pltpu
from jax.experimental.pallas import tpu_sc as plsc

_GAMMA = 2.0
_THRESH = 0.7
_B, _C, _H, _W = 4, 19, 512, 512
_N = _B * _H * _W
_NMIN = _N // 16
_HB = 64

_NC = 2
_NS = 16
_NW = _NC * _NS
_L = 16
_CHUNK = _N // _NW
_NBINS = 2048
_NLB = 2


def _ce_body(p_ref, t_ref, ce_ref, acc_ref):
    x = p_ref[...]
    t = t_ref[...]
    s = jnp.sum(jnp.exp(x), axis=0)
    cidx = jax.lax.broadcasted_iota(jnp.int32, x.shape, 0)
    tgt = jnp.sum(jnp.where(cidx == t[None, :, :], x, 0.0), axis=0)
    ce = jnp.maximum(jnp.log(s) - tgt, 0.0)
    ce_ref[...] = ce
    p = jnp.exp(-ce)
    focal = ((1.0 - p) ** 2) * ce
    hthr = -jnp.log(jnp.float32(_THRESH))
    hard = ce > hthr

    @pl.when(pl.program_id(0) == 0)
    def _init():
        acc_ref[0, 0] = 0.0
        acc_ref[1, 0] = 0.0
        acc_ref[2, 0] = 0.0

    acc_ref[0, 0] += jnp.sum(focal)
    acc_ref[1, 0] += jnp.sum(jnp.where(hard, ce, 0.0))
    acc_ref[2, 0] += jnp.sum(hard.astype(jnp.float32))


def _ce_pass(preds, targets):
    p2 = preds.reshape(_B * _C, _H, _W)
    t2 = targets.reshape(_B * _H, _W)
    rows_per_b = _H // _HB
    grid = (_B * rows_per_b,)

    def p_map(i):
        return (i // rows_per_b, i % rows_per_b, 0)

    def t_map(i):
        return (i, 0)

    ce, acc = pl.pallas_call(
        _ce_body,
        grid=grid,
        in_specs=[
            pl.BlockSpec((_C, _HB, _W), p_map),
            pl.BlockSpec((_HB, _W), t_map),
        ],
        out_specs=[
            pl.BlockSpec((_HB, _W), t_map),
            pl.BlockSpec((3, 1), lambda i: (0, 0), memory_space=pltpu.SMEM),
        ],
        out_shape=[
            jax.ShapeDtypeStruct((_B * _H, _W), jnp.float32),
            jax.ShapeDtypeStruct((3, 1), jnp.float32),
        ],
    )(p2, t2)
    return ce, acc


def _hist_level(level):
    mesh = plsc.VectorSubcoreMesh(core_axis_name="c", subcore_axis_name="s")
    scratch = [
        pltpu.VMEM((_CHUNK,), jnp.float32),
        pltpu.VMEM((_NLB * _NBINS,), jnp.float32),
    ]
    if _NLB > 1:
        scratch.append(pltpu.VMEM((_NBINS,), jnp.float32))
    if level == 2:
        scratch.append(pltpu.VMEM((_L,), jnp.int32))

    def body(ce_hbm, *rest):
        rest = list(rest)
        if level == 2:
            bsel_hbm = rest.pop(0)
        out_hbm, vals_v, hist_v = rest[:3]
        red_v = rest[3] if _NLB > 1 else hist_v
        bsel_v = rest[-1] if level == 2 else None
        wid = lax.axis_index("s") * _NC + lax.axis_index("c")
        pltpu.sync_copy(ce_hbm.at[pl.ds(wid * _CHUNK, _CHUNK)], vals_v)
        if level == 2:
            pltpu.sync_copy(bsel_hbm.at[0, pl.ds(0, _L)], bsel_v)
            bstar = bsel_v[...]

        zeros16 = jnp.zeros((_L,), jnp.float32)
        _ZU = 8

        def zbody(i, carry):
            for u in range(_ZU):
                hist_v[pl.ds((i * _ZU + u) * _L, _L)] = zeros16
            return carry

        lax.fori_loop(0, (_NLB * _NBINS) // (_L * _ZU), zbody, None)

        ones = jnp.ones((_L,), jnp.float32)
        lane_off = (lax.iota(jnp.int32, _L) % _NLB) * _NBINS
        _HU = 8

        def hbody(i, carry):
            for u in range(_HU):
                v = vals_v[pl.ds((i * _HU + u) * _L, _L)]
                bits = lax.bitcast_convert_type(v, jnp.int32)
                if level == 1:
                    b = lane_off + lax.shift_right_logical(bits, 21)
                    plsc.addupdate_scatter(hist_v, [b], ones)
                else:
                    b1 = lax.shift_right_logical(bits, 21)
                    msk = b1 == bstar
                    b = lane_off + jnp.bitwise_and(
                        lax.shift_right_logical(bits, 10), _NBINS - 1)
                    plsc.addupdate_scatter(hist_v, [b], ones, mask=msk)
            return carry

        lax.fori_loop(0, _CHUNK // (_L * _HU), hbody, None)

        if _NLB > 1:
            def rbody(j, carry):
                acc = hist_v[pl.ds(j * _L, _L)]
                for l in range(1, _NLB):
                    acc = acc + hist_v[pl.ds(l * _NBINS + j * _L, _L)]
                red_v[pl.ds(j * _L, _L)] = acc
                return carry

            lax.fori_loop(0, _NBINS // _L, rbody, None)
        pltpu.sync_copy(red_v, out_hbm.at[wid])

    return pl.kernel(
        body,
        out_type=jax.ShapeDtypeStruct((_NW, _NBINS), jnp.float32),
        mesh=mesh,
        scratch_types=scratch,
        compiler_params=pltpu.CompilerParams(needs_layout_passes=False),
    )


def _suffix_cumsum(x):
    n = x.shape[1]
    sh = 1
    while sh < n:
        x = x + jnp.concatenate(
            [x[:, sh:], jnp.zeros((1, sh), x.dtype)], axis=1)
        sh *= 2
    return x


def _mini1_body(h1_ref, bsel_ref, scal_ref):
    c1 = jnp.sum(h1_ref[...], axis=0, keepdims=True)
    rc1 = _suffix_cumsum(c1)
    kf = jnp.float32(_NMIN)
    bstar = jnp.sum((rc1 >= kf).astype(jnp.float32)) - 1.0
    bins = lax.broadcasted_iota(jnp.int32, (1, _NBINS), 1).astype(jnp.float32)
    c_above = jnp.sum(jnp.where(bins > bstar, c1, 0.0))
    r1 = kf - c_above
    bsel_ref[...] = jnp.broadcast_to(bstar.astype(jnp.int32), (8, 128))
    scal_ref[0, 0] = bstar
    scal_ref[1, 0] = r1


def _mini2_body(ce_ref, h2_ref, scal_ref, acc_ref, res_ref, smi_ref, smf_ref):
    @pl.when(pl.program_id(0) == 0)
    def _select():
        c2 = jnp.sum(h2_ref[...], axis=0, keepdims=True)
        rc2 = _suffix_cumsum(c2)
        r1 = scal_ref[1, 0]
        jstar = jnp.sum((rc2 >= r1).astype(jnp.float32)) - 1.0
        bins = lax.broadcasted_iota(jnp.int32, (1, _NBINS), 1).astype(jnp.float32)
        c_above2 = jnp.sum(jnp.where(bins > jstar, c2, 0.0))
        c_sub = jnp.sum(jnp.where(bins == jstar, c2, 0.0))
        sub_f = scal_ref[0, 0] * jnp.float32(_NBINS) + jstar
        smi_ref[0, 0] = sub_f.astype(jnp.int32)
        smf_ref[0, 0] = 0.0
        smf_ref[1, 0] = 0.0
        smf_ref[2, 0] = r1 - c_above2
        smf_ref[3, 0] = c_sub

    x = ce_ref[...]
    b10 = lax.shift_right_logical(lax.bitcast_convert_type(x, jnp.int32), 10)
    sub = smi_ref[0, 0]
    smf_ref[0, 0] += jnp.sum(jnp.where(b10 > sub, x, 0.0))
    smf_ref[1, 0] += jnp.sum(jnp.where(b10 == sub, x, 0.0))

    @pl.when(pl.program_id(0) == pl.num_programs(0) - 1)
    def _final():
        kf = jnp.float32(_NMIN)
        sum_hi, sum_sub = smf_ref[0, 0], smf_ref[1, 0]
        r2, c_sub = smf_ref[2, 0], smf_ref[3, 0]
        mean_topk = (sum_hi + r2 * sum_sub / c_sub) / kf
        focal = acc_ref[0, 0] / jnp.float32(_N)
        mean_hard = acc_ref[1, 0] / jnp.maximum(acc_ref[2, 0], 1.0)
        ohem = jnp.where(acc_ref[2, 0] < kf, mean_topk, mean_hard)
        res_ref[0, 0] = focal + ohem


_MINI2_ROWS = 512


def _ohem_select(ce, acc):
    ce_flat = ce.reshape(-1)
    out1 = _hist_level(1)(ce_flat)
    bsel, scal1 = pl.pallas_call(
        _mini1_body,
        grid=(1,),
        in_specs=[pl.BlockSpec((_NW, _NBINS), lambda i: (0, 0))],
        out_specs=[
            pl.BlockSpec((8, 128), lambda i: (0, 0)),
            pl.BlockSpec((2, 1), lambda i: (0, 0), memory_space=pltpu.SMEM),
        ],
        out_shape=[
            jax.ShapeDtypeStruct((8, 128), jnp.int32),
            jax.ShapeDtypeStruct((2, 1), jnp.float32),
        ],
    )(out1)
    out2 = _hist_level(2)(ce_flat, bsel)
    res = pl.pallas_call(
        _mini2_body,
        grid=(_B * _H // _MINI2_ROWS,),
        in_specs=[
            pl.BlockSpec((_MINI2_ROWS, _W), lambda i: (i, 0)),
            pl.BlockSpec((_NW, _NBINS), lambda i: (0, 0)),
            pl.BlockSpec((2, 1), lambda i: (0, 0), memory_space=pltpu.SMEM),
            pl.BlockSpec((3, 1), lambda i: (0, 0), memory_space=pltpu.SMEM),
        ],
        out_specs=pl.BlockSpec((1, 1), lambda i: (0, 0),
                               memory_space=pltpu.SMEM),
        out_shape=jax.ShapeDtypeStruct((1, 1), jnp.float32),
        scratch_shapes=[
            pltpu.SMEM((1, 1), jnp.int32),
            pltpu.SMEM((4, 1), jnp.float32),
        ],
    )(ce, out2, scal1, acc)
    return res


def kernel(preds, targets):
    ce, acc = _ce_pass(preds, targets)
    res = _ohem_select(ce, acc)
    return res[0, 0]


# speedup vs baseline: 1.0331x; 1.0331x over previous
import functools

import jax
import jax.numpy as jnp
from jax import lax
from jax.experimental import pallas as pl
from jax.experimental.pallas import tpu as pltpu
from jax.experimental.pallas import tpu_sc as plsc

_GAMMA = 2.0
_THRESH = 0.7
_B, _C, _H, _W = 4, 19, 512, 512
_N = _B * _H * _W
_NMIN = _N // 16
_HB = 64

_NC = 2
_NS = 16
_NW = _NC * _NS
_L = 16
_CHUNK = _N // _NW
_NBINS = 2048
_NLB = 1


def _ce_body(p_ref, t_ref, ce_ref, acc_ref):
    x = p_ref[...]
    t = t_ref[...]
    s = jnp.sum(jnp.exp(x), axis=0)
    cidx = jax.lax.broadcasted_iota(jnp.int32, x.shape, 0)
    tgt = jnp.sum(jnp.where(cidx == t[None, :, :], x, 0.0), axis=0)
    ce = jnp.maximum(jnp.log(s) - tgt, 0.0)
    ce_ref[...] = ce
    p = jnp.exp(-ce)
    focal = ((1.0 - p) ** 2) * ce
    hthr = -jnp.log(jnp.float32(_THRESH))
    hard = ce > hthr

    @pl.when(pl.program_id(0) == 0)
    def _init():
        acc_ref[0, 0] = 0.0
        acc_ref[1, 0] = 0.0
        acc_ref[2, 0] = 0.0

    acc_ref[0, 0] += jnp.sum(focal)
    acc_ref[1, 0] += jnp.sum(jnp.where(hard, ce, 0.0))
    acc_ref[2, 0] += jnp.sum(hard.astype(jnp.float32))


def _ce_pass(preds, targets):
    p2 = preds.reshape(_B * _C, _H, _W)
    t2 = targets.reshape(_B * _H, _W)
    rows_per_b = _H // _HB
    grid = (_B * rows_per_b,)

    def p_map(i):
        return (i // rows_per_b, i % rows_per_b, 0)

    def t_map(i):
        return (i, 0)

    ce, acc = pl.pallas_call(
        _ce_body,
        grid=grid,
        in_specs=[
            pl.BlockSpec((_C, _HB, _W), p_map),
            pl.BlockSpec((_HB, _W), t_map),
        ],
        out_specs=[
            pl.BlockSpec((_HB, _W), t_map),
            pl.BlockSpec((3, 1), lambda i: (0, 0), memory_space=pltpu.SMEM),
        ],
        out_shape=[
            jax.ShapeDtypeStruct((_B * _H, _W), jnp.float32),
            jax.ShapeDtypeStruct((3, 1), jnp.float32),
        ],
    )(p2, t2)
    return ce, acc


def _hist_level(level):
    mesh = plsc.VectorSubcoreMesh(core_axis_name="c", subcore_axis_name="s")
    scratch = [
        pltpu.VMEM((_CHUNK,), jnp.float32),
        pltpu.VMEM((_NLB * _NBINS,), jnp.float32),
    ]
    if _NLB > 1:
        scratch.append(pltpu.VMEM((_NBINS,), jnp.float32))
    if level == 2:
        scratch.append(pltpu.VMEM((_L,), jnp.int32))

    def body(ce_hbm, *rest):
        rest = list(rest)
        if level == 2:
            bsel_hbm = rest.pop(0)
        out_hbm, vals_v, hist_v = rest[:3]
        red_v = rest[3] if _NLB > 1 else hist_v
        bsel_v = rest[-1] if level == 2 else None
        wid = lax.axis_index("s") * _NC + lax.axis_index("c")
        pltpu.sync_copy(ce_hbm.at[pl.ds(wid * _CHUNK, _CHUNK)], vals_v)
        if level == 2:
            pltpu.sync_copy(bsel_hbm.at[0, pl.ds(0, _L)], bsel_v)
            bstar = bsel_v[...]

        zeros16 = jnp.zeros((_L,), jnp.float32)
        _ZU = 8

        def zbody(i, carry):
            for u in range(_ZU):
                hist_v[pl.ds((i * _ZU + u) * _L, _L)] = zeros16
            return carry

        lax.fori_loop(0, (_NLB * _NBINS) // (_L * _ZU), zbody, None)

        ones = jnp.ones((_L,), jnp.float32)
        lane_off = (lax.iota(jnp.int32, _L) % _NLB) * _NBINS
        _HU = 16

        def hbody(i, carry):
            for u in range(_HU):
                v = vals_v[pl.ds((i * _HU + u) * _L, _L)]
                bits = lax.bitcast_convert_type(v, jnp.int32)
                if level == 1:
                    b = lane_off + lax.shift_right_logical(bits, 21)
                    plsc.addupdate_scatter(hist_v, [b], ones)
                else:
                    b1 = lax.shift_right_logical(bits, 21)
                    msk = b1 == bstar
                    b = lane_off + jnp.bitwise_and(
                        lax.shift_right_logical(bits, 10), _NBINS - 1)
                    plsc.addupdate_scatter(hist_v, [b], ones, mask=msk)
            return carry

        lax.fori_loop(0, _CHUNK // (_L * _HU), hbody, None)

        if _NLB > 1:
            def rbody(j, carry):
                acc = hist_v[pl.ds(j * _L, _L)]
                for l in range(1, _NLB):
                    acc = acc + hist_v[pl.ds(l * _NBINS + j * _L, _L)]
                red_v[pl.ds(j * _L, _L)] = acc
                return carry

            lax.fori_loop(0, _NBINS // _L, rbody, None)
        pltpu.sync_copy(red_v, out_hbm.at[wid])

    return pl.kernel(
        body,
        out_type=jax.ShapeDtypeStruct((_NW, _NBINS), jnp.float32),
        mesh=mesh,
        scratch_types=scratch,
        compiler_params=pltpu.CompilerParams(needs_layout_passes=False),
    )


def _suffix_cumsum(x):
    n = x.shape[1]
    sh = 1
    while sh < n:
        x = x + jnp.concatenate(
            [x[:, sh:], jnp.zeros((1, sh), x.dtype)], axis=1)
        sh *= 2
    return x


def _mini1_body(h1_ref, bsel_ref, scal_ref):
    c1 = jnp.sum(h1_ref[...], axis=0, keepdims=True)
    rc1 = _suffix_cumsum(c1)
    kf = jnp.float32(_NMIN)
    bstar = jnp.sum((rc1 >= kf).astype(jnp.float32)) - 1.0
    bins = lax.broadcasted_iota(jnp.int32, (1, _NBINS), 1).astype(jnp.float32)
    c_above = jnp.sum(jnp.where(bins > bstar, c1, 0.0))
    r1 = kf - c_above
    bsel_ref[...] = jnp.broadcast_to(bstar.astype(jnp.int32), (8, 128))
    scal_ref[0, 0] = bstar
    scal_ref[1, 0] = r1


def _mini2_body(ce_ref, h2_ref, scal_ref, acc_ref, res_ref, smi_ref, smf_ref):
    @pl.when(pl.program_id(0) == 0)
    def _select():
        c2 = jnp.sum(h2_ref[...], axis=0, keepdims=True)
        rc2 = _suffix_cumsum(c2)
        r1 = scal_ref[1, 0]
        jstar = jnp.sum((rc2 >= r1).astype(jnp.float32)) - 1.0
        bins = lax.broadcasted_iota(jnp.int32, (1, _NBINS), 1).astype(jnp.float32)
        c_above2 = jnp.sum(jnp.where(bins > jstar, c2, 0.0))
        c_sub = jnp.sum(jnp.where(bins == jstar, c2, 0.0))
        sub_f = scal_ref[0, 0] * jnp.float32(_NBINS) + jstar
        smi_ref[0, 0] = sub_f.astype(jnp.int32)
        smf_ref[0, 0] = 0.0
        smf_ref[1, 0] = 0.0
        smf_ref[2, 0] = r1 - c_above2
        smf_ref[3, 0] = c_sub

    x = ce_ref[...]
    b10 = lax.shift_right_logical(lax.bitcast_convert_type(x, jnp.int32), 10)
    sub = smi_ref[0, 0]
    smf_ref[0, 0] += jnp.sum(jnp.where(b10 > sub, x, 0.0))
    smf_ref[1, 0] += jnp.sum(jnp.where(b10 == sub, x, 0.0))

    @pl.when(pl.program_id(0) == pl.num_programs(0) - 1)
    def _final():
        kf = jnp.float32(_NMIN)
        sum_hi, sum_sub = smf_ref[0, 0], smf_ref[1, 0]
        r2, c_sub = smf_ref[2, 0], smf_ref[3, 0]
        mean_topk = (sum_hi + r2 * sum_sub / c_sub) / kf
        focal = acc_ref[0, 0] / jnp.float32(_N)
        mean_hard = acc_ref[1, 0] / jnp.maximum(acc_ref[2, 0], 1.0)
        ohem = jnp.where(acc_ref[2, 0] < kf, mean_topk, mean_hard)
        res_ref[0, 0] = focal + ohem


_MINI2_ROWS = 512


def _ohem_select(ce, acc):
    ce_flat = ce.reshape(-1)
    out1 = _hist_level(1)(ce_flat)
    bsel, scal1 = pl.pallas_call(
        _mini1_body,
        grid=(1,),
        in_specs=[pl.BlockSpec((_NW, _NBINS), lambda i: (0, 0))],
        out_specs=[
            pl.BlockSpec((8, 128), lambda i: (0, 0)),
            pl.BlockSpec((2, 1), lambda i: (0, 0), memory_space=pltpu.SMEM),
        ],
        out_shape=[
            jax.ShapeDtypeStruct((8, 128), jnp.int32),
            jax.ShapeDtypeStruct((2, 1), jnp.float32),
        ],
    )(out1)
    out2 = _hist_level(2)(ce_flat, bsel)
    res = pl.pallas_call(
        _mini2_body,
        grid=(_B * _H // _MINI2_ROWS,),
        in_specs=[
            pl.BlockSpec((_MINI2_ROWS, _W), lambda i: (i, 0)),
            pl.BlockSpec((_NW, _NBINS), lambda i: (0, 0)),
            pl.BlockSpec((2, 1), lambda i: (0, 0), memory_space=pltpu.SMEM),
            pl.BlockSpec((3, 1), lambda i: (0, 0), memory_space=pltpu.SMEM),
        ],
        out_specs=pl.BlockSpec((1, 1), lambda i: (0, 0),
                               memory_space=pltpu.SMEM),
        out_shape=jax.ShapeDtypeStruct((1, 1), jnp.float32),
        scratch_shapes=[
            pltpu.SMEM((1, 1), jnp.int32),
            pltpu.SMEM((4, 1), jnp.float32),
        ],
    )(ce, out2, scal1, acc)
    return res


def kernel(preds, targets):
    ce, acc = _ce_pass(preds, targets)
    res = _ohem_select(ce, acc)
    return res[0, 0]


# speedup vs baseline: 1.1146x; 1.0789x over previous
import functools

import jax
import jax.numpy as jnp
from jax import lax
from jax.experimental import pallas as pl
from jax.experimental.pallas import tpu as pltpu
from jax.experimental.pallas import tpu_sc as plsc

_GAMMA = 2.0
_THRESH = 0.7
_B, _C, _H, _W = 4, 19, 512, 512
_N = _B * _H * _W
_NMIN = _N // 16
_HB = 128

_NC = 2
_NS = 16
_NW = _NC * _NS
_L = 16
_CHUNK = _N // _NW
_NBINS = 2048
_NLB = 1


def _ce_body(p_ref, t_ref, ce_ref, acc_ref):
    x = p_ref[...]
    t = t_ref[...]
    s = jnp.sum(jnp.exp(x), axis=0)
    cidx = jax.lax.broadcasted_iota(jnp.int32, x.shape, 0)
    tgt = jnp.sum(jnp.where(cidx == t[None, :, :], x, 0.0), axis=0)
    ce = jnp.maximum(jnp.log(s) - tgt, 0.0)
    ce_ref[...] = ce
    p = jnp.exp(-ce)
    focal = ((1.0 - p) ** 2) * ce
    hthr = -jnp.log(jnp.float32(_THRESH))
    hard = ce > hthr

    @pl.when(pl.program_id(0) == 0)
    def _init():
        acc_ref[0, 0] = 0.0
        acc_ref[1, 0] = 0.0
        acc_ref[2, 0] = 0.0

    acc_ref[0, 0] += jnp.sum(focal)
    acc_ref[1, 0] += jnp.sum(jnp.where(hard, ce, 0.0))
    acc_ref[2, 0] += jnp.sum(hard.astype(jnp.float32))


def _ce_pass(preds, targets):
    p2 = preds.reshape(_B * _C, _H, _W)
    t2 = targets.reshape(_B * _H, _W)
    rows_per_b = _H // _HB
    grid = (_B * rows_per_b,)

    def p_map(i):
        return (i // rows_per_b, i % rows_per_b, 0)

    def t_map(i):
        return (i, 0)

    ce, acc = pl.pallas_call(
        _ce_body,
        grid=grid,
        in_specs=[
            pl.BlockSpec((_C, _HB, _W), p_map),
            pl.BlockSpec((_HB, _W), t_map),
        ],
        out_specs=[
            pl.BlockSpec((_HB, _W), t_map),
            pl.BlockSpec((3, 1), lambda i: (0, 0), memory_space=pltpu.SMEM),
        ],
        out_shape=[
            jax.ShapeDtypeStruct((_B * _H, _W), jnp.float32),
            jax.ShapeDtypeStruct((3, 1), jnp.float32),
        ],
    )(p2, t2)
    return ce, acc


def _hist_level(level):
    mesh = plsc.VectorSubcoreMesh(core_axis_name="c", subcore_axis_name="s")
    scratch = [
        pltpu.VMEM((_CHUNK,), jnp.float32),
        pltpu.VMEM((_NLB * _NBINS,), jnp.float32),
    ]
    if _NLB > 1:
        scratch.append(pltpu.VMEM((_NBINS,), jnp.float32))
    if level == 2:
        scratch.append(pltpu.VMEM((_L,), jnp.int32))

    def body(ce_hbm, *rest):
        rest = list(rest)
        if level == 2:
            bsel_hbm = rest.pop(0)
        out_hbm, vals_v, hist_v = rest[:3]
        red_v = rest[3] if _NLB > 1 else hist_v
        bsel_v = rest[-1] if level == 2 else None
        wid = lax.axis_index("s") * _NC + lax.axis_index("c")
        pltpu.sync_copy(ce_hbm.at[pl.ds(wid * _CHUNK, _CHUNK)], vals_v)
        if level == 2:
            pltpu.sync_copy(bsel_hbm.at[0, pl.ds(0, _L)], bsel_v)
            bstar = bsel_v[...]

        zeros16 = jnp.zeros((_L,), jnp.float32)
        _ZU = 8

        def zbody(i, carry):
            for u in range(_ZU):
                hist_v[pl.ds((i * _ZU + u) * _L, _L)] = zeros16
            return carry

        lax.fori_loop(0, (_NLB * _NBINS) // (_L * _ZU), zbody, None)

        ones = jnp.ones((_L,), jnp.float32)
        lane_off = (lax.iota(jnp.int32, _L) % _NLB) * _NBINS
        _HU = 16

        def hbody(i, carry):
            for u in range(_HU):
                v = vals_v[pl.ds((i * _HU + u) * _L, _L)]
                bits = lax.bitcast_convert_type(v, jnp.int32)
                if level == 1:
                    b = lane_off + lax.shift_right_logical(bits, 21)
                    plsc.addupdate_scatter(hist_v, [b], ones)
                else:
                    b1 = lax.shift_right_logical(bits, 21)
                    msk = b1 == bstar
                    b = lane_off + jnp.bitwise_and(
                        lax.shift_right_logical(bits, 10), _NBINS - 1)
                    plsc.addupdate_scatter(hist_v, [b], ones, mask=msk)
            return carry

        lax.fori_loop(0, _CHUNK // (_L * _HU), hbody, None)

        if _NLB > 1:
            def rbody(j, carry):
                acc = hist_v[pl.ds(j * _L, _L)]
                for l in range(1, _NLB):
                    acc = acc + hist_v[pl.ds(l * _NBINS + j * _L, _L)]
                red_v[pl.ds(j * _L, _L)] = acc
                return carry

            lax.fori_loop(0, _NBINS // _L, rbody, None)
        pltpu.sync_copy(red_v, out_hbm.at[wid])

    return pl.kernel(
        body,
        out_type=jax.ShapeDtypeStruct((_NW, _NBINS), jnp.float32),
        mesh=mesh,
        scratch_types=scratch,
        compiler_params=pltpu.CompilerParams(needs_layout_passes=False),
    )


def _suffix_cumsum(x):
    n = x.shape[1]
    sh = 1
    while sh < n:
        x = x + jnp.concatenate(
            [x[:, sh:], jnp.zeros((1, sh), x.dtype)], axis=1)
        sh *= 2
    return x


def _mini1_body(h1_ref, bsel_ref, scal_ref):
    c1 = jnp.sum(h1_ref[...], axis=0, keepdims=True)
    rc1 = _suffix_cumsum(c1)
    kf = jnp.float32(_NMIN)
    bstar = jnp.sum((rc1 >= kf).astype(jnp.float32)) - 1.0
    bins = lax.broadcasted_iota(jnp.int32, (1, _NBINS), 1).astype(jnp.float32)
    c_above = jnp.sum(jnp.where(bins > bstar, c1, 0.0))
    r1 = kf - c_above
    bsel_ref[...] = jnp.broadcast_to(bstar.astype(jnp.int32), (8, 128))
    scal_ref[0, 0] = bstar
    scal_ref[1, 0] = r1


def _mini2_body(ce_ref, h2_ref, scal_ref, acc_ref, res_ref, smi_ref, smf_ref):
    @pl.when(pl.program_id(0) == 0)
    def _select():
        c2 = jnp.sum(h2_ref[...], axis=0, keepdims=True)
        rc2 = _suffix_cumsum(c2)
        r1 = scal_ref[1, 0]
        jstar = jnp.sum((rc2 >= r1).astype(jnp.float32)) - 1.0
        bins = lax.broadcasted_iota(jnp.int32, (1, _NBINS), 1).astype(jnp.float32)
        c_above2 = jnp.sum(jnp.where(bins > jstar, c2, 0.0))
        c_sub = jnp.sum(jnp.where(bins == jstar, c2, 0.0))
        sub_f = scal_ref[0, 0] * jnp.float32(_NBINS) + jstar
        smi_ref[0, 0] = sub_f.astype(jnp.int32)
        smf_ref[0, 0] = 0.0
        smf_ref[1, 0] = 0.0
        smf_ref[2, 0] = r1 - c_above2
        smf_ref[3, 0] = c_sub

    x = ce_ref[...]
    b10 = lax.shift_right_logical(lax.bitcast_convert_type(x, jnp.int32), 10)
    sub = smi_ref[0, 0]
    smf_ref[0, 0] += jnp.sum(jnp.where(b10 > sub, x, 0.0))
    smf_ref[1, 0] += jnp.sum(jnp.where(b10 == sub, x, 0.0))

    @pl.when(pl.program_id(0) == pl.num_programs(0) - 1)
    def _final():
        kf = jnp.float32(_NMIN)
        sum_hi, sum_sub = smf_ref[0, 0], smf_ref[1, 0]
        r2, c_sub = smf_ref[2, 0], smf_ref[3, 0]
        mean_topk = (sum_hi + r2 * sum_sub / c_sub) / kf
        focal = acc_ref[0, 0] / jnp.float32(_N)
        mean_hard = acc_ref[1, 0] / jnp.maximum(acc_ref[2, 0], 1.0)
        ohem = jnp.where(acc_ref[2, 0] < kf, mean_topk, mean_hard)
        res_ref[0, 0] = focal + ohem


_MINI2_ROWS = 512


def _ohem_select(ce, acc):
    ce_flat = ce.reshape(-1)
    out1 = _hist_level(1)(ce_flat)
    bsel, scal1 = pl.pallas_call(
        _mini1_body,
        grid=(1,),
        in_specs=[pl.BlockSpec((_NW, _NBINS), lambda i: (0, 0))],
        out_specs=[
            pl.BlockSpec((8, 128), lambda i: (0, 0)),
            pl.BlockSpec((2, 1), lambda i: (0, 0), memory_space=pltpu.SMEM),
        ],
        out_shape=[
            jax.ShapeDtypeStruct((8, 128), jnp.int32),
            jax.ShapeDtypeStruct((2, 1), jnp.float32),
        ],
    )(out1)
    out2 = _hist_level(2)(ce_flat, bsel)
    res = pl.pallas_call(
        _mini2_body,
        grid=(_B * _H // _MINI2_ROWS,),
        in_specs=[
            pl.BlockSpec((_MINI2_ROWS, _W), lambda i: (i, 0)),
            pl.BlockSpec((_NW, _NBINS), lambda i: (0, 0)),
            pl.BlockSpec((2, 1), lambda i: (0, 0), memory_space=pltpu.SMEM),
            pl.BlockSpec((3, 1), lambda i: (0, 0), memory_space=pltpu.SMEM),
        ],
        out_specs=pl.BlockSpec((1, 1), lambda i: (0, 0),
                               memory_space=pltpu.SMEM),
        out_shape=jax.ShapeDtypeStruct((1, 1), jnp.float32),
        scratch_shapes=[
            pltpu.SMEM((1, 1), jnp.int32),
            pltpu.SMEM((4, 1), jnp.float32),
        ],
    )(ce, out2, scal1, acc)
    return res


def kernel(preds, targets):
    ce, acc = _ce_pass(preds, targets)
    res = _ohem_select(ce, acc)
    return res[0, 0]


# speedup vs baseline: 1.1348x; 1.0181x over previous
import functools

import jax
import jax.numpy as jnp
from jax import lax
from jax.experimental import pallas as pl
from jax.experimental.pallas import tpu as pltpu
from jax.experimental.pallas import tpu_sc as plsc

_GAMMA = 2.0
_THRESH = 0.7
_B, _C, _H, _W = 4, 19, 512, 512
_N = _B * _H * _W
_NMIN = _N // 16
_HB = 256

_NC = 2
_NS = 16
_NW = _NC * _NS
_L = 16
_CHUNK = _N // _NW
_NBINS = 2048
_NLB = 1


def _ce_body(p_ref, t_ref, ce_ref, acc_ref):
    x = p_ref[...]
    t = t_ref[...]
    s = jnp.sum(jnp.exp(x), axis=0)
    cidx = jax.lax.broadcasted_iota(jnp.int32, x.shape, 0)
    tgt = jnp.sum(jnp.where(cidx == t[None, :, :], x, 0.0), axis=0)
    ce = jnp.maximum(jnp.log(s) - tgt, 0.0)
    ce_ref[...] = ce
    p = jnp.exp(-ce)
    focal = ((1.0 - p) ** 2) * ce
    hthr = -jnp.log(jnp.float32(_THRESH))
    hard = ce > hthr

    @pl.when(pl.program_id(0) == 0)
    def _init():
        acc_ref[0, 0] = 0.0
        acc_ref[1, 0] = 0.0
        acc_ref[2, 0] = 0.0

    acc_ref[0, 0] += jnp.sum(focal)
    acc_ref[1, 0] += jnp.sum(jnp.where(hard, ce, 0.0))
    acc_ref[2, 0] += jnp.sum(hard.astype(jnp.float32))


def _ce_pass(preds, targets):
    p2 = preds.reshape(_B * _C, _H, _W)
    t2 = targets.reshape(_B * _H, _W)
    rows_per_b = _H // _HB
    grid = (_B * rows_per_b,)

    def p_map(i):
        return (i // rows_per_b, i % rows_per_b, 0)

    def t_map(i):
        return (i, 0)

    ce, acc = pl.pallas_call(
        _ce_body,
        grid=grid,
        in_specs=[
            pl.BlockSpec((_C, _HB, _W), p_map),
            pl.BlockSpec((_HB, _W), t_map),
        ],
        out_specs=[
            pl.BlockSpec((_HB, _W), t_map),
            pl.BlockSpec((3, 1), lambda i: (0, 0), memory_space=pltpu.SMEM),
        ],
        out_shape=[
            jax.ShapeDtypeStruct((_B * _H, _W), jnp.float32),
            jax.ShapeDtypeStruct((3, 1), jnp.float32),
        ],
    )(p2, t2)
    return ce, acc


def _hist_level(level):
    mesh = plsc.VectorSubcoreMesh(core_axis_name="c", subcore_axis_name="s")
    scratch = [
        pltpu.VMEM((_CHUNK,), jnp.float32),
        pltpu.VMEM((_NLB * _NBINS,), jnp.float32),
    ]
    if _NLB > 1:
        scratch.append(pltpu.VMEM((_NBINS,), jnp.float32))
    if level == 2:
        scratch.append(pltpu.VMEM((_L,), jnp.int32))

    def body(ce_hbm, *rest):
        rest = list(rest)
        if level == 2:
            bsel_hbm = rest.pop(0)
        out_hbm, vals_v, hist_v = rest[:3]
        red_v = rest[3] if _NLB > 1 else hist_v
        bsel_v = rest[-1] if level == 2 else None
        wid = lax.axis_index("s") * _NC + lax.axis_index("c")
        pltpu.sync_copy(ce_hbm.at[pl.ds(wid * _CHUNK, _CHUNK)], vals_v)
        if level == 2:
            pltpu.sync_copy(bsel_hbm.at[0, pl.ds(0, _L)], bsel_v)
            bstar = bsel_v[...]

        zeros16 = jnp.zeros((_L,), jnp.float32)
        _ZU = 8

        def zbody(i, carry):
            for u in range(_ZU):
                hist_v[pl.ds((i * _ZU + u) * _L, _L)] = zeros16
            return carry

        lax.fori_loop(0, (_NLB * _NBINS) // (_L * _ZU), zbody, None)

        ones = jnp.ones((_L,), jnp.float32)
        lane_off = (lax.iota(jnp.int32, _L) % _NLB) * _NBINS
        _HU = 16

        def hbody(i, carry):
            for u in range(_HU):
                v = vals_v[pl.ds((i * _HU + u) * _L, _L)]
                bits = lax.bitcast_convert_type(v, jnp.int32)
                if level == 1:
                    b = lane_off + lax.shift_right_logical(bits, 21)
                    plsc.addupdate_scatter(hist_v, [b], ones)
                else:
                    b1 = lax.shift_right_logical(bits, 21)
                    msk = b1 == bstar
                    b = lane_off + jnp.bitwise_and(
                        lax.shift_right_logical(bits, 10), _NBINS - 1)
                    plsc.addupdate_scatter(hist_v, [b], ones, mask=msk)
            return carry

        lax.fori_loop(0, _CHUNK // (_L * _HU), hbody, None)

        if _NLB > 1:
            def rbody(j, carry):
                acc = hist_v[pl.ds(j * _L, _L)]
                for l in range(1, _NLB):
                    acc = acc + hist_v[pl.ds(l * _NBINS + j * _L, _L)]
                red_v[pl.ds(j * _L, _L)] = acc
                return carry

            lax.fori_loop(0, _NBINS // _L, rbody, None)
        pltpu.sync_copy(red_v, out_hbm.at[wid])

    return pl.kernel(
        body,
        out_type=jax.ShapeDtypeStruct((_NW, _NBINS), jnp.float32),
        mesh=mesh,
        scratch_types=scratch,
        compiler_params=pltpu.CompilerParams(needs_layout_passes=False),
    )


def _suffix_cumsum(x):
    n = x.shape[1]
    sh = 1
    while sh < n:
        x = x + jnp.concatenate(
            [x[:, sh:], jnp.zeros((1, sh), x.dtype)], axis=1)
        sh *= 2
    return x


def _mini1_body(h1_ref, bsel_ref, scal_ref):
    c1 = jnp.sum(h1_ref[...], axis=0, keepdims=True)
    rc1 = _suffix_cumsum(c1)
    kf = jnp.float32(_NMIN)
    bstar = jnp.sum((rc1 >= kf).astype(jnp.float32)) - 1.0
    bins = lax.broadcasted_iota(jnp.int32, (1, _NBINS), 1).astype(jnp.float32)
    c_above = jnp.sum(jnp.where(bins > bstar, c1, 0.0))
    r1 = kf - c_above
    bsel_ref[...] = jnp.broadcast_to(bstar.astype(jnp.int32), (8, 128))
    scal_ref[0, 0] = bstar
    scal_ref[1, 0] = r1


def _mini2_body(ce_ref, h2_ref, scal_ref, acc_ref, res_ref, smi_ref, smf_ref):
    @pl.when(pl.program_id(0) == 0)
    def _select():
        c2 = jnp.sum(h2_ref[...], axis=0, keepdims=True)
        rc2 = _suffix_cumsum(c2)
        r1 = scal_ref[1, 0]
        jstar = jnp.sum((rc2 >= r1).astype(jnp.float32)) - 1.0
        bins = lax.broadcasted_iota(jnp.int32, (1, _NBINS), 1).astype(jnp.float32)
        c_above2 = jnp.sum(jnp.where(bins > jstar, c2, 0.0))
        c_sub = jnp.sum(jnp.where(bins == jstar, c2, 0.0))
        sub_f = scal_ref[0, 0] * jnp.float32(_NBINS) + jstar
        smi_ref[0, 0] = sub_f.astype(jnp.int32)
        smf_ref[0, 0] = 0.0
        smf_ref[1, 0] = 0.0
        smf_ref[2, 0] = r1 - c_above2
        smf_ref[3, 0] = c_sub

    x = ce_ref[...]
    b10 = lax.shift_right_logical(lax.bitcast_convert_type(x, jnp.int32), 10)
    sub = smi_ref[0, 0]
    smf_ref[0, 0] += jnp.sum(jnp.where(b10 > sub, x, 0.0))
    smf_ref[1, 0] += jnp.sum(jnp.where(b10 == sub, x, 0.0))

    @pl.when(pl.program_id(0) == pl.num_programs(0) - 1)
    def _final():
        kf = jnp.float32(_NMIN)
        sum_hi, sum_sub = smf_ref[0, 0], smf_ref[1, 0]
        r2, c_sub = smf_ref[2, 0], smf_ref[3, 0]
        mean_topk = (sum_hi + r2 * sum_sub / c_sub) / kf
        focal = acc_ref[0, 0] / jnp.float32(_N)
        mean_hard = acc_ref[1, 0] / jnp.maximum(acc_ref[2, 0], 1.0)
        ohem = jnp.where(acc_ref[2, 0] < kf, mean_topk, mean_hard)
        res_ref[0, 0] = focal + ohem


_MINI2_ROWS = 512


def _ohem_select(ce, acc):
    ce_flat = ce.reshape(-1)
    out1 = _hist_level(1)(ce_flat)
    bsel, scal1 = pl.pallas_call(
        _mini1_body,
        grid=(1,),
        in_specs=[pl.BlockSpec((_NW, _NBINS), lambda i: (0, 0))],
        out_specs=[
            pl.BlockSpec((8, 128), lambda i: (0, 0)),
            pl.BlockSpec((2, 1), lambda i: (0, 0), memory_space=pltpu.SMEM),
        ],
        out_shape=[
            jax.ShapeDtypeStruct((8, 128), jnp.int32),
            jax.ShapeDtypeStruct((2, 1), jnp.float32),
        ],
    )(out1)
    out2 = _hist_level(2)(ce_flat, bsel)
    res = pl.pallas_call(
        _mini2_body,
        grid=(_B * _H // _MINI2_ROWS,),
        in_specs=[
            pl.BlockSpec((_MINI2_ROWS, _W), lambda i: (i, 0)),
            pl.BlockSpec((_NW, _NBINS), lambda i: (0, 0)),
            pl.BlockSpec((2, 1), lambda i: (0, 0), memory_space=pltpu.SMEM),
            pl.BlockSpec((3, 1), lambda i: (0, 0), memory_space=pltpu.SMEM),
        ],
        out_specs=pl.BlockSpec((1, 1), lambda i: (0, 0),
                               memory_space=pltpu.SMEM),
        out_shape=jax.ShapeDtypeStruct((1, 1), jnp.float32),
        scratch_shapes=[
            pltpu.SMEM((1, 1), jnp.int32),
            pltpu.SMEM((4, 1), jnp.float32),
        ],
    )(ce, out2, scal1, acc)
    return res


def kernel(preds, targets):
    ce, acc = _ce_pass(preds, targets)
    res = _ohem_select(ce, acc)
    return res[0, 0]


# speedup vs baseline: 1.1352x; 1.0003x over previous
import jax
import jax.numpy as jnp
from jax import lax
from jax.experimental import pallas as pl
from jax.experimental.pallas import tpu as pltpu
from jax.experimental.pallas import tpu_sc as plsc

_THRESH = 0.7
_B, _C, _H, _W = 4, 19, 512, 512
_N = _B * _H * _W
_NMIN = _N // 16
_HB = 256

_NC = 2
_NS = 16
_NW = _NC * _NS
_L = 16
_CHUNK = _N // _NW
_NBINS = 2048
_NLB = 1


def _ce_body(p_ref, t_ref, ce_ref, acc_ref):
    x = p_ref[...]
    t = t_ref[...]
    s = jnp.sum(jnp.exp(x), axis=0)
    cidx = jax.lax.broadcasted_iota(jnp.int32, x.shape, 0)
    tgt = jnp.sum(jnp.where(cidx == t[None, :, :], x, 0.0), axis=0)
    ce = jnp.maximum(jnp.log(s) - tgt, 0.0)
    ce_ref[...] = ce
    p = jnp.exp(-ce)
    focal = ((1.0 - p) ** 2) * ce
    hthr = -jnp.log(jnp.float32(_THRESH))
    hard = ce > hthr

    @pl.when(pl.program_id(0) == 0)
    def _init():
        acc_ref[0, 0] = 0.0
        acc_ref[1, 0] = 0.0
        acc_ref[2, 0] = 0.0

    acc_ref[0, 0] += jnp.sum(focal)
    acc_ref[1, 0] += jnp.sum(jnp.where(hard, ce, 0.0))
    acc_ref[2, 0] += jnp.sum(hard.astype(jnp.float32))


def _ce_pass(preds, targets):
    p2 = preds.reshape(_B * _C, _H, _W)
    t2 = targets.reshape(_B * _H, _W)
    rows_per_b = _H // _HB
    grid = (_B * rows_per_b,)

    def p_map(i):
        return (i // rows_per_b, i % rows_per_b, 0)

    def t_map(i):
        return (i, 0)

    ce, acc = pl.pallas_call(
        _ce_body,
        grid=grid,
        in_specs=[
            pl.BlockSpec((_C, _HB, _W), p_map),
            pl.BlockSpec((_HB, _W), t_map),
        ],
        out_specs=[
            pl.BlockSpec((_HB, _W), t_map),
            pl.BlockSpec((3, 1), lambda i: (0, 0), memory_space=pltpu.SMEM),
        ],
        out_shape=[
            jax.ShapeDtypeStruct((_B * _H, _W), jnp.float32),
            jax.ShapeDtypeStruct((3, 1), jnp.float32),
        ],
    )(p2, t2)
    return ce, acc


def _hist_level(level):
    mesh = plsc.VectorSubcoreMesh(core_axis_name="c", subcore_axis_name="s")
    scratch = [
        pltpu.VMEM((_CHUNK,), jnp.float32),
        pltpu.VMEM((_NLB * _NBINS,), jnp.float32),
    ]
    if _NLB > 1:
        scratch.append(pltpu.VMEM((_NBINS,), jnp.float32))
    if level == 2:
        scratch.append(pltpu.VMEM((_L,), jnp.int32))

    def body(ce_hbm, *rest):
        rest = list(rest)
        if level == 2:
            bsel_hbm = rest.pop(0)
        out_hbm, vals_v, hist_v = rest[:3]
        red_v = rest[3] if _NLB > 1 else hist_v
        bsel_v = rest[-1] if level == 2 else None
        wid = lax.axis_index("s") * _NC + lax.axis_index("c")
        pltpu.sync_copy(ce_hbm.at[pl.ds(wid * _CHUNK, _CHUNK)], vals_v)
        if level == 2:
            pltpu.sync_copy(bsel_hbm.at[0, pl.ds(0, _L)], bsel_v)
            bstar = bsel_v[...]

        zeros16 = jnp.zeros((_L,), jnp.float32)
        _ZU = 8

        def zbody(i, carry):
            for u in range(_ZU):
                hist_v[pl.ds((i * _ZU + u) * _L, _L)] = zeros16
            return carry

        lax.fori_loop(0, (_NLB * _NBINS) // (_L * _ZU), zbody, None)

        ones = jnp.ones((_L,), jnp.float32)
        lane_off = (lax.iota(jnp.int32, _L) % _NLB) * _NBINS
        _HU = 16

        def hbody(i, carry):
            for u in range(_HU):
                v = vals_v[pl.ds((i * _HU + u) * _L, _L)]
                bits = lax.bitcast_convert_type(v, jnp.int32)
                if level == 1:
                    b = lane_off + lax.shift_right_logical(bits, 21)
                    plsc.addupdate_scatter(hist_v, [b], ones)
                else:
                    b1 = lax.shift_right_logical(bits, 21)
                    msk = b1 == bstar
                    b = lane_off + jnp.bitwise_and(
                        lax.shift_right_logical(bits, 10), _NBINS - 1)
                    plsc.addupdate_scatter(hist_v, [b], ones, mask=msk)
            return carry

        lax.fori_loop(0, _CHUNK // (_L * _HU), hbody, None)

        if _NLB > 1:
            def rbody(j, carry):
                acc = hist_v[pl.ds(j * _L, _L)]
                for l in range(1, _NLB):
                    acc = acc + hist_v[pl.ds(l * _NBINS + j * _L, _L)]
                red_v[pl.ds(j * _L, _L)] = acc
                return carry

            lax.fori_loop(0, _NBINS // _L, rbody, None)
        pltpu.sync_copy(red_v, out_hbm.at[wid])

    return pl.kernel(
        body,
        out_type=jax.ShapeDtypeStruct((_NW, _NBINS), jnp.float32),
        mesh=mesh,
        scratch_types=scratch,
        compiler_params=pltpu.CompilerParams(needs_layout_passes=False),
    )


def _suffix_cumsum(x):
    n = x.shape[1]
    sh = 1
    while sh < n:
        x = x + jnp.concatenate(
            [x[:, sh:], jnp.zeros((1, sh), x.dtype)], axis=1)
        sh *= 2
    return x


def _mini1_body(h1_ref, bsel_ref, scal_ref):
    c1 = jnp.sum(h1_ref[...], axis=0, keepdims=True)
    rc1 = _suffix_cumsum(c1)
    kf = jnp.float32(_NMIN)
    bstar = jnp.sum((rc1 >= kf).astype(jnp.float32)) - 1.0
    bins = lax.broadcasted_iota(jnp.int32, (1, _NBINS), 1).astype(jnp.float32)
    c_above = jnp.sum(jnp.where(bins > bstar, c1, 0.0))
    r1 = kf - c_above
    bsel_ref[...] = jnp.broadcast_to(bstar.astype(jnp.int32), (8, 128))
    scal_ref[0, 0] = bstar
    scal_ref[1, 0] = r1


def _mini2_body(ce_ref, h2_ref, scal_ref, acc_ref, res_ref, smi_ref, smf_ref):
    @pl.when(pl.program_id(0) == 0)
    def _select():
        c2 = jnp.sum(h2_ref[...], axis=0, keepdims=True)
        rc2 = _suffix_cumsum(c2)
        r1 = scal_ref[1, 0]
        jstar = jnp.sum((rc2 >= r1).astype(jnp.float32)) - 1.0
        bins = lax.broadcasted_iota(jnp.int32, (1, _NBINS), 1).astype(jnp.float32)
        c_above2 = jnp.sum(jnp.where(bins > jstar, c2, 0.0))
        c_sub = jnp.sum(jnp.where(bins == jstar, c2, 0.0))
        sub_f = scal_ref[0, 0] * jnp.float32(_NBINS) + jstar
        smi_ref[0, 0] = sub_f.astype(jnp.int32)
        smf_ref[0, 0] = 0.0
        smf_ref[1, 0] = 0.0
        smf_ref[2, 0] = r1 - c_above2
        smf_ref[3, 0] = c_sub

    x = ce_ref[...]
    b10 = lax.shift_right_logical(lax.bitcast_convert_type(x, jnp.int32), 10)
    sub = smi_ref[0, 0]
    smf_ref[0, 0] += jnp.sum(jnp.where(b10 > sub, x, 0.0))
    smf_ref[1, 0] += jnp.sum(jnp.where(b10 == sub, x, 0.0))

    @pl.when(pl.program_id(0) == pl.num_programs(0) - 1)
    def _final():
        kf = jnp.float32(_NMIN)
        sum_hi, sum_sub = smf_ref[0, 0], smf_ref[1, 0]
        r2, c_sub = smf_ref[2, 0], smf_ref[3, 0]
        mean_topk = (sum_hi + r2 * sum_sub / c_sub) / kf
        focal = acc_ref[0, 0] / jnp.float32(_N)
        mean_hard = acc_ref[1, 0] / jnp.maximum(acc_ref[2, 0], 1.0)
        ohem = jnp.where(acc_ref[2, 0] < kf, mean_topk, mean_hard)
        res_ref[0, 0] = focal + ohem


_MINI2_ROWS = 512


def _ohem_select(ce, acc):
    ce_flat = ce.reshape(-1)
    out1 = _hist_level(1)(ce_flat)
    bsel, scal1 = pl.pallas_call(
        _mini1_body,
        grid=(1,),
        in_specs=[pl.BlockSpec((_NW, _NBINS), lambda i: (0, 0))],
        out_specs=[
            pl.BlockSpec((8, 128), lambda i: (0, 0)),
            pl.BlockSpec((2, 1), lambda i: (0, 0), memory_space=pltpu.SMEM),
        ],
        out_shape=[
            jax.ShapeDtypeStruct((8, 128), jnp.int32),
            jax.ShapeDtypeStruct((2, 1), jnp.float32),
        ],
    )(out1)
    out2 = _hist_level(2)(ce_flat, bsel)
    res = pl.pallas_call(
        _mini2_body,
        grid=(_B * _H // _MINI2_ROWS,),
        in_specs=[
            pl.BlockSpec((_MINI2_ROWS, _W), lambda i: (i, 0)),
            pl.BlockSpec((_NW, _NBINS), lambda i: (0, 0)),
            pl.BlockSpec((2, 1), lambda i: (0, 0), memory_space=pltpu.SMEM),
            pl.BlockSpec((3, 1), lambda i: (0, 0), memory_space=pltpu.SMEM),
        ],
        out_specs=pl.BlockSpec((1, 1), lambda i: (0, 0),
                               memory_space=pltpu.SMEM),
        out_shape=jax.ShapeDtypeStruct((1, 1), jnp.float32),
        scratch_shapes=[
            pltpu.SMEM((1, 1), jnp.int32),
            pltpu.SMEM((4, 1), jnp.float32),
        ],
    )(ce, out2, scal1, acc)
    return res


def kernel(preds, targets):
    ce, acc = _ce_pass(preds, targets)
    res = _ohem_select(ce, acc)
    return res[0, 0]
